# Initial kernel scaffold; baseline (speedup 1.0000x reference)
import jax
import jax.numpy as jnp
from jax.experimental import pallas as pl


def kernel(x, edge_index, batch, Wg, Wf):
    raise NotImplementedError("write your pallas kernel here")


# speedup vs baseline: 100.5514x; 100.5514x over previous
import functools

import jax
import jax.numpy as jnp
import numpy as np
from jax import lax
from jax.experimental import pallas as pl
from jax.experimental.pallas import tpu as pltpu
from jax.experimental.pallas import tpu_sc as plsc

_N = 100000
_E = 1600000
_D_OUT = 128
_WIDTHS = [12, 6, 5, 6, 5, 1, 1, 1, 1]
_OFFS = np.concatenate([[0], np.cumsum(_WIDTHS)])
_NG = len(_WIDTHS)
_DIN = int(_OFFS[-1])
_GW = 16

_NPAD = 100352
_EROWS = 12544
_EPAD = _EROWS * 128
_ROWS_PER_TILE = _EROWS // 32
_K = 8
_ZROWS = _NPAD // 16

_GBLK = 2048
_OBLK = 1024


def _gates_body(x_ref, wg_ref, g_ref):
    g_ref[...] = jnp.dot(x_ref[...], wg_ref[...],
                         preferred_element_type=jnp.float32)


def _out_body(x_ref, p0_ref, p1_ref, exp_ref, wf_ref, out_ref, attn_ref):
    logits = p0_ref[0] + p1_ref[0]
    lane = lax.broadcasted_iota(jnp.int32, logits.shape, 1)
    valid = lane < _NG
    lm = jnp.where(valid, logits, -1e30)
    m = jnp.max(lm, axis=1, keepdims=True)
    e = jnp.where(valid, jnp.exp(lm - m), 0.0)
    s = jnp.sum(e, axis=1, keepdims=True)
    attn = e / s
    attn_ref[...] = attn
    ax = jnp.dot(attn, exp_ref[...],
                 preferred_element_type=jnp.float32)
    out_ref[...] = jnp.dot(x_ref[...] * ax, wf_ref[...],
                           preferred_element_type=jnp.float32)


def _segment_sum_sc(g, src2, dst2, zrows):
    mesh = plsc.VectorSubcoreMesh(core_axis_name="c", subcore_axis_name="s")

    @functools.partial(
        pl.kernel,
        out_type=jax.ShapeDtypeStruct((2, _NPAD, _GW), jnp.float32),
        mesh=mesh,
        scratch_types=[
            pltpu.VMEM_SHARED((_NPAD, _GW), jnp.float32),
            pltpu.VMEM((_K, 128), jnp.int32),
            pltpu.VMEM((_K, 128), jnp.int32),
            pltpu.VMEM((_K, 128, _GW), jnp.float32),
            pltpu.SemaphoreType.DMA,
        ],
        compiler_params=pltpu.CompilerParams(use_tc_tiling_on_sc=False),
    )
    def seg_sum(g_hbm, src_hbm, dst_hbm, z_hbm, out_hbm,
                acc, sidx, didx, rows, sem):
        cid = lax.axis_index("c")
        sid = lax.axis_index("s")
        zbase = sid * _ZROWS
        pltpu.sync_copy(z_hbm, acc.at[pl.ds(zbase, _ZROWS)])
        plsc.subcore_barrier()

        tile_row0 = (cid * 16 + sid) * _ROWS_PER_TILE

        def step(gi, carry):
            rb = tile_row0 + gi * _K
            pltpu.sync_copy(src_hbm.at[pl.ds(rb, _K)], sidx)
            pltpu.sync_copy(dst_hbm.at[pl.ds(rb, _K)], didx)
            cps = [pltpu.async_copy(g_hbm.at[sidx.at[j]], rows.at[j], sem)
                   for j in range(_K)]
            for cp in cps:
                cp.wait()
            for j in range(_K):
                pltpu.sync_copy(rows.at[j], acc.at[didx.at[j]], add=True)
            return carry

        lax.fori_loop(0, _ROWS_PER_TILE // _K, step, 0)
        plsc.subcore_barrier()
        pltpu.sync_copy(acc.at[pl.ds(zbase, _ZROWS)],
                        out_hbm.at[cid, pl.ds(zbase, _ZROWS)])

    return seg_sum(g, src2, dst2, zrows)


def kernel(x, edge_index, batch, Wg, Wf):
    del batch

    wg_block = jnp.zeros((_DIN, _GW), jnp.float32)
    for i in range(_NG):
        wg_block = wg_block.at[_OFFS[i]:_OFFS[i + 1], i].set(Wg[i][:, 0])
    wf_block = jnp.concatenate(Wf, axis=0)

    expand = np.zeros((_GW, _DIN), np.float32)
    for i in range(_NG):
        expand[i, _OFFS[i]:_OFFS[i + 1]] = 1.0
    expand = jnp.asarray(expand)

    x_pad = jnp.concatenate(
        [x, jnp.zeros((_NPAD - _N, _DIN), jnp.float32)], axis=0)
    src = jnp.concatenate(
        [edge_index[0], jnp.full((_EPAD - _E,), _N, jnp.int32)])
    dst = jnp.concatenate(
        [edge_index[1], jnp.full((_EPAD - _E,), _N, jnp.int32)])
    src2 = src.reshape(_EROWS, 128)
    dst2 = dst.reshape(_EROWS, 128)
    zrows = jnp.zeros((_ZROWS, _GW), jnp.float32)

    g = pl.pallas_call(
        _gates_body,
        grid=(_NPAD // _GBLK,),
        in_specs=[
            pl.BlockSpec((_GBLK, _DIN), lambda i: (i, 0)),
            pl.BlockSpec((_DIN, _GW), lambda i: (0, 0)),
        ],
        out_specs=pl.BlockSpec((_GBLK, _GW), lambda i: (i, 0)),
        out_shape=jax.ShapeDtypeStruct((_NPAD, _GW), jnp.float32),
    )(x_pad, wg_block)

    partial = _segment_sum_sc(g, src2, dst2, zrows)

    out_full, attn16 = pl.pallas_call(
        _out_body,
        grid=(_NPAD // _OBLK,),
        in_specs=[
            pl.BlockSpec((_OBLK, _DIN), lambda i: (i, 0)),
            pl.BlockSpec((1, _OBLK, _GW), lambda i: (0, i, 0)),
            pl.BlockSpec((1, _OBLK, _GW), lambda i: (1, i, 0)),
            pl.BlockSpec((_GW, _DIN), lambda i: (0, 0)),
            pl.BlockSpec((_DIN, _D_OUT), lambda i: (0, 0)),
        ],
        out_specs=[
            pl.BlockSpec((_OBLK, _D_OUT), lambda i: (i, 0)),
            pl.BlockSpec((_OBLK, _GW), lambda i: (i, 0)),
        ],
        out_shape=[
            jax.ShapeDtypeStruct((_NPAD, _D_OUT), jnp.float32),
            jax.ShapeDtypeStruct((_NPAD, _GW), jnp.float32),
        ],
    )(x_pad, partial, partial, expand, wf_block)

    out = out_full[:_N]
    attention = attn16[:_N, :_NG][:, :, None]
    return out, attention


# speedup vs baseline: 120.7789x; 1.2012x over previous
import functools

import jax
import jax.numpy as jnp
import numpy as np
from jax import lax
from jax.experimental import pallas as pl
from jax.experimental.pallas import tpu as pltpu
from jax.experimental.pallas import tpu_sc as plsc

_N = 100000
_E = 1600000
_D_OUT = 128
_WIDTHS = [12, 6, 5, 6, 5, 1, 1, 1, 1]
_OFFS = np.concatenate([[0], np.cumsum(_WIDTHS)])
_NG = len(_WIDTHS)
_DIN = int(_OFFS[-1])
_GW = 16

_EROWS = _E // 128
_K = 8
_RPT = 392
_TAILBASE = 31 * _RPT
_ZROWS = _N // 16

_GBLK = 2048
_OBLK = 1024


def _gates_body(x_ref, wg_ref, g_ref):
    g_ref[...] = jnp.dot(x_ref[...], wg_ref[...],
                         preferred_element_type=jnp.float32)


def _out_body(x_ref, p0_ref, p1_ref, exp_ref, wf_ref, out_ref, attn_ref):
    logits = p0_ref[0] + p1_ref[0]
    lane = lax.broadcasted_iota(jnp.int32, logits.shape, 1)
    valid = lane < _NG
    lm = jnp.where(valid, logits, -1e30)
    m = jnp.max(lm, axis=1, keepdims=True)
    e = jnp.where(valid, jnp.exp(lm - m), 0.0)
    s = jnp.sum(e, axis=1, keepdims=True)
    attn = e / s
    attn_ref[...] = attn[:, :_NG]
    ax = jnp.dot(attn, exp_ref[...],
                 preferred_element_type=jnp.float32)
    out_ref[...] = jnp.dot(x_ref[...] * ax, wf_ref[...],
                           preferred_element_type=jnp.float32)


def _segment_sum_sc(g, edge3, zrows):
    mesh = plsc.VectorSubcoreMesh(core_axis_name="c", subcore_axis_name="s")

    @functools.partial(
        pl.kernel,
        out_type=jax.ShapeDtypeStruct((2, _N, _GW), jnp.float32),
        mesh=mesh,
        scratch_types=[
            pltpu.VMEM_SHARED((_N, _GW), jnp.float32),
            pltpu.VMEM((1, _K, 128), jnp.int32),
            pltpu.VMEM((1, _K, 128), jnp.int32),
            pltpu.VMEM((1, _K, 128, _GW), jnp.float32),
            pltpu.SemaphoreType.DMA,
        ],
        compiler_params=pltpu.CompilerParams(use_tc_tiling_on_sc=False),
    )
    def seg_sum(g_hbm, e_hbm, z_hbm, out_hbm, acc, sidx, didx, rows, sem):
        cid = lax.axis_index("c")
        sid = lax.axis_index("s")
        zbase = sid * _ZROWS
        pltpu.sync_copy(z_hbm, acc.at[pl.ds(zbase, _ZROWS)])
        plsc.subcore_barrier()

        wid = cid * 16 + sid
        row0 = wid * _RPT
        ntrips = jnp.where(wid < 31, _RPT // _K, 43)

        def step(gi, carry):
            rb = row0 + gi * _K
            pltpu.sync_copy(e_hbm.at[0, pl.ds(rb, _K)], sidx.at[0])
            pltpu.sync_copy(e_hbm.at[1, pl.ds(rb, _K)], didx.at[0])
            cps = [pltpu.async_copy(g_hbm.at[sidx.at[0, j]],
                                    rows.at[0, j], sem) for j in range(_K)]
            for cp in cps:
                cp.wait()
            for j in range(_K):
                pltpu.sync_copy(rows.at[0, j], acc.at[didx.at[0, j]],
                                add=True)
            return carry

        lax.fori_loop(0, ntrips, step, 0)

        @pl.when(wid == 31)
        def _():
            pltpu.sync_copy(e_hbm.at[0, pl.ds(_TAILBASE + 344, 4)],
                            sidx.at[0, pl.ds(0, 4)])
            pltpu.sync_copy(e_hbm.at[1, pl.ds(_TAILBASE + 344, 4)],
                            didx.at[0, pl.ds(0, 4)])
            cps = [pltpu.async_copy(g_hbm.at[sidx.at[0, j]],
                                    rows.at[0, j], sem) for j in range(4)]
            for cp in cps:
                cp.wait()
            for j in range(4):
                pltpu.sync_copy(rows.at[0, j], acc.at[didx.at[0, j]],
                                add=True)

        plsc.subcore_barrier()
        pltpu.sync_copy(acc.at[pl.ds(zbase, _ZROWS)],
                        out_hbm.at[cid, pl.ds(zbase, _ZROWS)])

    return seg_sum(g, edge3, zrows)


def kernel(x, edge_index, batch, Wg, Wf):
    del batch

    wg_block = jnp.zeros((_DIN, _GW), jnp.float32)
    for i in range(_NG):
        wg_block = wg_block.at[_OFFS[i]:_OFFS[i + 1], i].set(Wg[i][:, 0])
    wf_block = jnp.concatenate(Wf, axis=0)

    expand = np.zeros((_GW, _DIN), np.float32)
    for i in range(_NG):
        expand[i, _OFFS[i]:_OFFS[i + 1]] = 1.0
    expand = jnp.asarray(expand)

    edge3 = edge_index.reshape(2, _EROWS, 128)
    zrows = jnp.zeros((_ZROWS, _GW), jnp.float32)

    g = pl.pallas_call(
        _gates_body,
        grid=(pl.cdiv(_N, _GBLK),),
        in_specs=[
            pl.BlockSpec((_GBLK, _DIN), lambda i: (i, 0)),
            pl.BlockSpec((_DIN, _GW), lambda i: (0, 0)),
        ],
        out_specs=pl.BlockSpec((_GBLK, _GW), lambda i: (i, 0)),
        out_shape=jax.ShapeDtypeStruct((_N, _GW), jnp.float32),
    )(x, wg_block)

    partial = _segment_sum_sc(g, edge3, zrows)

    out, attn9 = pl.pallas_call(
        _out_body,
        grid=(pl.cdiv(_N, _OBLK),),
        in_specs=[
            pl.BlockSpec((_OBLK, _DIN), lambda i: (i, 0)),
            pl.BlockSpec((1, _OBLK, _GW), lambda i: (0, i, 0)),
            pl.BlockSpec((1, _OBLK, _GW), lambda i: (1, i, 0)),
            pl.BlockSpec((_GW, _DIN), lambda i: (0, 0)),
            pl.BlockSpec((_DIN, _D_OUT), lambda i: (0, 0)),
        ],
        out_specs=[
            pl.BlockSpec((_OBLK, _D_OUT), lambda i: (i, 0)),
            pl.BlockSpec((_OBLK, _NG), lambda i: (i, 0)),
        ],
        out_shape=[
            jax.ShapeDtypeStruct((_N, _D_OUT), jnp.float32),
            jax.ShapeDtypeStruct((_N, _NG), jnp.float32),
        ],
    )(x, partial, partial, expand, wf_block)

    return out, attn9[:, :, None]


# speedup vs baseline: 125.8421x; 1.0419x over previous
import functools

import jax
import jax.numpy as jnp
import numpy as np
from jax import lax
from jax.experimental import pallas as pl
from jax.experimental.pallas import tpu as pltpu
from jax.experimental.pallas import tpu_sc as plsc

_N = 100000
_E = 1600000
_D_OUT = 128
_WIDTHS = [12, 6, 5, 6, 5, 1, 1, 1, 1]
_OFFS = np.concatenate([[0], np.cumsum(_WIDTHS)])
_NG = len(_WIDTHS)
_DIN = int(_OFFS[-1])
_GW = 16

_EROWS = _E // 128
_K = 6
_RPT = 390
_NTRIPS = _RPT // _K
_TAILBASE = 32 * _RPT
_ZROWS = _N // 16

_GBLK = 2048
_OBLK = 1024


def _gates_body(x_ref, wg_ref, g_ref):
    g_ref[...] = jnp.dot(x_ref[...], wg_ref[...],
                         preferred_element_type=jnp.float32)


def _out_body(x_ref, p0_ref, p1_ref, exp_ref, wf_ref, out_ref, attn_ref):
    logits = p0_ref[0] + p1_ref[0]
    lane = lax.broadcasted_iota(jnp.int32, logits.shape, 1)
    valid = lane < _NG
    lm = jnp.where(valid, logits, -1e30)
    m = jnp.max(lm, axis=1, keepdims=True)
    e = jnp.where(valid, jnp.exp(lm - m), 0.0)
    s = jnp.sum(e, axis=1, keepdims=True)
    attn = e / s
    attn_ref[...] = attn[:, :_NG]
    ax = jnp.dot(attn, exp_ref[...],
                 preferred_element_type=jnp.float32)
    out_ref[...] = jnp.dot(x_ref[...] * ax, wf_ref[...],
                           preferred_element_type=jnp.float32)


def _segment_sum_sc(g, edge3, zrows):
    mesh = plsc.VectorSubcoreMesh(core_axis_name="c", subcore_axis_name="s")

    @functools.partial(
        pl.kernel,
        out_type=jax.ShapeDtypeStruct((2, _N, _GW), jnp.float32),
        mesh=mesh,
        scratch_types=[
            pltpu.VMEM_SHARED((_N, _GW), jnp.float32),
            pltpu.VMEM((2, _K, 128), jnp.int32),
            pltpu.VMEM((2, _K, 128), jnp.int32),
            pltpu.VMEM((2, _K, 128, _GW), jnp.float32),
            pltpu.SemaphoreType.DMA,
        ],
        compiler_params=pltpu.CompilerParams(use_tc_tiling_on_sc=False),
    )
    def seg_sum(g_hbm, e_hbm, z_hbm, out_hbm, acc, sidx, didx, rows, sem):
        cid = lax.axis_index("c")
        sid = lax.axis_index("s")
        zbase = sid * _ZROWS
        pltpu.sync_copy(z_hbm, acc.at[pl.ds(zbase, _ZROWS)])
        plsc.subcore_barrier()

        wid = cid * 16 + sid
        row0 = wid * _RPT

        def stage(par, rb):
            pltpu.sync_copy(e_hbm.at[0, pl.ds(rb, _K)], sidx.at[par])
            pltpu.sync_copy(e_hbm.at[1, pl.ds(rb, _K)], didx.at[par])
            for j in range(_K):
                pltpu.async_copy(g_hbm.at[sidx.at[par, j]],
                                 rows.at[par, j], sem)

        stage(0, row0)

        def step(gi, carry):
            par = lax.rem(gi, 2)
            for j in range(_K):
                pltpu.make_async_copy(g_hbm.at[sidx.at[par, j]],
                                      rows.at[par, j], sem).wait()

            @pl.when(gi + 1 < _NTRIPS)
            def _():
                stage(1 - par, row0 + (gi + 1) * _K)

            for j in range(_K):
                pltpu.sync_copy(rows.at[par, j], acc.at[didx.at[par, j]],
                                add=True)
            return carry

        lax.fori_loop(0, _NTRIPS, step, 0)

        @pl.when(wid < 20)
        def _():
            pltpu.sync_copy(e_hbm.at[0, pl.ds(_TAILBASE + wid, 1)],
                            sidx.at[0, pl.ds(0, 1)])
            pltpu.sync_copy(e_hbm.at[1, pl.ds(_TAILBASE + wid, 1)],
                            didx.at[0, pl.ds(0, 1)])
            pltpu.async_copy(g_hbm.at[sidx.at[0, 0]],
                             rows.at[0, 0], sem).wait()
            pltpu.sync_copy(rows.at[0, 0], acc.at[didx.at[0, 0]],
                            add=True)

        plsc.subcore_barrier()
        pltpu.sync_copy(acc.at[pl.ds(zbase, _ZROWS)],
                        out_hbm.at[cid, pl.ds(zbase, _ZROWS)])

    return seg_sum(g, edge3, zrows)


def kernel(x, edge_index, batch, Wg, Wf):
    del batch

    wg_block = jnp.zeros((_DIN, _GW), jnp.float32)
    for i in range(_NG):
        wg_block = wg_block.at[_OFFS[i]:_OFFS[i + 1], i].set(Wg[i][:, 0])
    wf_block = jnp.concatenate(Wf, axis=0)

    expand = np.zeros((_GW, _DIN), np.float32)
    for i in range(_NG):
        expand[i, _OFFS[i]:_OFFS[i + 1]] = 1.0
    expand = jnp.asarray(expand)

    edge3 = edge_index.reshape(2, _EROWS, 128)
    zrows = jnp.zeros((_ZROWS, _GW), jnp.float32)

    g = pl.pallas_call(
        _gates_body,
        grid=(pl.cdiv(_N, _GBLK),),
        in_specs=[
            pl.BlockSpec((_GBLK, _DIN), lambda i: (i, 0)),
            pl.BlockSpec((_DIN, _GW), lambda i: (0, 0)),
        ],
        out_specs=pl.BlockSpec((_GBLK, _GW), lambda i: (i, 0)),
        out_shape=jax.ShapeDtypeStruct((_N, _GW), jnp.float32),
    )(x, wg_block)

    partial = _segment_sum_sc(g, edge3, zrows)

    out, attn9 = pl.pallas_call(
        _out_body,
        grid=(pl.cdiv(_N, _OBLK),),
        in_specs=[
            pl.BlockSpec((_OBLK, _DIN), lambda i: (i, 0)),
            pl.BlockSpec((1, _OBLK, _GW), lambda i: (0, i, 0)),
            pl.BlockSpec((1, _OBLK, _GW), lambda i: (1, i, 0)),
            pl.BlockSpec((_GW, _DIN), lambda i: (0, 0)),
            pl.BlockSpec((_DIN, _D_OUT), lambda i: (0, 0)),
        ],
        out_specs=[
            pl.BlockSpec((_OBLK, _D_OUT), lambda i: (i, 0)),
            pl.BlockSpec((_OBLK, _NG), lambda i: (i, 0)),
        ],
        out_shape=[
            jax.ShapeDtypeStruct((_N, _D_OUT), jnp.float32),
            jax.ShapeDtypeStruct((_N, _NG), jnp.float32),
        ],
    )(x, partial, partial, expand, wf_block)

    return out, attn9[:, :, None]
